# baseline (speedup 1.0000x reference)
import jax
import jax.numpy as jnp
from jax import lax
from jax.experimental import pallas as pl
from jax.experimental.pallas import tpu as pltpu, tpu_sc as plsc

B, S, DEPTH = 1024, 50, 1000
NW = 32
DC = 125
CPS = DEPTH // DC
NCHUNK = S * CPS
GROUPS = B // 16


def _sc_body(idx_hbm, zeros_hbm, out_hbm, idx_v, buf):
    wid = lax.axis_index("s") * 2 + lax.axis_index("c")
    pltpu.sync_copy(zeros_hbm, buf)
    iota = lax.iota(jnp.int32, 16)
    ones = jnp.full((16,), 1.0, jnp.float32)
    zeros = jnp.zeros((16,), jnp.float32)
    nmine = (NCHUNK - wid + NW - 1) // NW

    def scatter(d0, val):
        for g in range(GROUPS):
            b16 = iota + g * 16
            idxg = idx_v[pl.ds(g * 16, 16)]
            dloc = idxg - d0
            mask = (dloc >= 0) & (dloc < DC)
            dloc = jnp.minimum(jnp.maximum(dloc, 0), DC - 1)
            plsc.store_scatter(buf, [dloc, b16], val, mask=mask)

    def chunk(i, carry):
        cid = wid + i * NW
        s = cid // CPS
        d0 = (cid % CPS) * DC
        pltpu.sync_copy(idx_hbm.at[pl.ds(s * B, B)], idx_v)
        scatter(d0, ones)
        pltpu.sync_copy(buf, out_hbm.at[s].at[pl.ds(d0, DC)])
        scatter(d0, zeros)
        return carry

    lax.fori_loop(0, nmine, chunk, 0)


def kernel(inputs):
    idx_t = inputs.astype(jnp.int32).T.reshape(S * B)
    zblock = jnp.zeros((DC, B), jnp.float32)
    mesh = plsc.VectorSubcoreMesh(core_axis_name="c", subcore_axis_name="s")
    k = pl.kernel(
        _sc_body,
        out_type=jax.ShapeDtypeStruct((S, DEPTH, B), jnp.float32),
        mesh=mesh,
        compiler_params=pltpu.CompilerParams(use_tc_tiling_on_sc=False, needs_layout_passes=False),
        scratch_types=[
            pltpu.VMEM((B,), jnp.int32),
            pltpu.VMEM((DC, B), jnp.float32),
        ],
    )
    out_t = k(idx_t, zblock)
    return out_t.transpose(2, 0, 1)


# speedup vs baseline: 5.0348x; 5.0348x over previous
import jax
import jax.numpy as jnp
from jax import lax
from jax.experimental import pallas as pl

B, S, DEPTH = 1024, 50, 1000
BS = 2


def _onehot_t_body(idx_ref, out_ref):
    rows = idx_ref[0]
    d_iota = lax.broadcasted_iota(jnp.int32, (BS, DEPTH, B), 1)
    out_ref[...] = (rows[:, None, :] == d_iota).astype(jnp.float32)


def kernel(inputs):
    idx_t = inputs.astype(jnp.int32).T.reshape(S // BS, BS, B)
    out_t = pl.pallas_call(
        _onehot_t_body,
        grid=(S // BS,),
        in_specs=[pl.BlockSpec((1, BS, B), lambda i: (i, 0, 0))],
        out_specs=pl.BlockSpec((BS, DEPTH, B), lambda i: (i, 0, 0)),
        out_shape=jax.ShapeDtypeStruct((S, DEPTH, B), jnp.float32),
    )(idx_t)
    return out_t.reshape(S, DEPTH, B).transpose(2, 0, 1)
